# baseline (speedup 1.0000x reference)
import jax
import jax.numpy as jnp
from jax import lax
from jax.experimental import pallas as pl
from jax.experimental.pallas import tpu as pltpu
from jax.experimental.pallas import tpu_sc as plsc

BATCH = 16384
EMB = 64
CDIM = 128
NC = 2
NS = 16
NW = NC * NS
BPW = BATCH // NW
CHUNK = 128
NCHUNK = BPW // CHUNK
NSLOT = 2


def _make_gather_body(width):
    def body(tab, idx_hbm, out, idx_v, buf, sg0, sg1, sw0, sw1):
        sem_g = [sg0, sg1]
        sem_w = [sw0, sw1]
        wid = lax.axis_index("s") * NC + lax.axis_index("c")
        base = wid * BPW
        pltpu.sync_copy(idx_hbm.at[pl.ds(wid * NCHUNK, NCHUNK)], idx_v)
        gh = [None] * NCHUNK
        wh = [None] * NCHUNK

        def start_wb(k):
            return pltpu.async_copy(
                buf.at[k % NSLOT], out.at[pl.ds(base + k * CHUNK, CHUNK)],
                sem_w[k % NSLOT])

        for k in range(NCHUNK):
            if k >= NSLOT:
                wh[k - NSLOT].wait()
            gh[k] = pltpu.async_copy(tab.at[idx_v.at[k]], buf.at[k % NSLOT],
                                     sem_g[k % NSLOT])
            if k >= 1:
                gh[k - 1].wait()
                wh[k - 1] = start_wb(k - 1)
        gh[NCHUNK - 1].wait()
        wh[NCHUNK - 1] = start_wb(NCHUNK - 1)
        for k in range(NCHUNK - NSLOT, NCHUNK):
            wh[k].wait()
    return body


def _make_gather(rows, width):
    mesh = plsc.VectorSubcoreMesh(core_axis_name="c", subcore_axis_name="s")
    return pl.kernel(
        _make_gather_body(width),
        out_type=jax.ShapeDtypeStruct((BATCH, width), jnp.float32),
        mesh=mesh,
        scratch_types=[
            pltpu.VMEM((NCHUNK, CHUNK), jnp.int32),
            pltpu.VMEM((NSLOT, CHUNK, width), jnp.float32),
        ] + [pltpu.SemaphoreType.DMA] * (2 * NSLOT),
        compiler_params=pltpu.CompilerParams(use_tc_tiling_on_sc=False),
    )


@jax.jit
def _sc_gather_u(user_emb, iu):
    return _make_gather(100000, EMB)(user_emb, iu)


@jax.jit
def _sc_gather_c(item_content, ii):
    return _make_gather(100000, CDIM)(item_content, ii)


def _tc_body(u_ref, c_ref, w_ref, b_ref, o_ref):
    meta = lax.dot_general(c_ref[...], w_ref[...],
                           (((1,), (0,)), ((), ())),
                           preferred_element_type=jnp.float32)
    meta = meta + b_ref[...]
    o_ref[...] = jnp.sum(u_ref[...] * meta, axis=1)[None, None, :]


@jax.jit
def _tc_compute(u_g, c_g, Wt5, b5):
    blk = 2048
    grid = BATCH // blk
    out = pl.pallas_call(
        _tc_body,
        grid=(grid,),
        in_specs=[
            pl.BlockSpec((blk, EMB), lambda i: (i, 0)),
            pl.BlockSpec((blk, CDIM), lambda i: (i, 0)),
            pl.BlockSpec((CDIM, EMB), lambda i: (0, 0)),
            pl.BlockSpec((1, EMB), lambda i: (0, 0)),
        ],
        out_specs=pl.BlockSpec((1, 1, blk), lambda i: (i, 0, 0)),
        out_shape=jax.ShapeDtypeStruct((grid, 1, blk), jnp.float32),
    )(u_g, c_g, Wt5, b5)
    return out.reshape(BATCH)


def kernel(batch_u, batch_i, user_emb, item_emb, item_content, W, b):
    iu = batch_u.astype(jnp.int32).reshape(BATCH // CHUNK, CHUNK)
    ii = batch_i.astype(jnp.int32).reshape(BATCH // CHUNK, CHUNK)
    c_g = _sc_gather_c(item_content, ii)
    u_g = _sc_gather_u(user_emb, iu)
    Wt5 = W.T / 5.0
    b5 = (b / 5.0).reshape(1, EMB)
    return _tc_compute(u_g, c_g, Wt5, b5)


# speedup vs baseline: 1.0275x; 1.0275x over previous
import jax
import jax.numpy as jnp
from jax import lax
from jax.experimental import pallas as pl
from jax.experimental.pallas import tpu as pltpu
from jax.experimental.pallas import tpu_sc as plsc

BATCH = 16384
EMB = 64
CDIM = 128
NC = 2
NS = 16
NW = NC * NS
BPW = BATCH // NW
CHUNK = 128
NCHUNK = BPW // CHUNK
NSLOT = 2


def _make_gather_body(width):
    def body(tab, idx_hbm, out, idx_v, buf, sg0, sg1, sw0, sw1):
        sem_g = [sg0, sg1]
        sem_w = [sw0, sw1]
        wid = lax.axis_index("s") * NC + lax.axis_index("c")
        base = wid * BPW
        pltpu.sync_copy(idx_hbm.at[pl.ds(wid * NCHUNK, NCHUNK)], idx_v)
        gh = [None] * NCHUNK
        wh = [None] * NCHUNK

        def start_wb(k):
            return pltpu.async_copy(
                buf.at[k % NSLOT], out.at[pl.ds(base + k * CHUNK, CHUNK)],
                sem_w[k % NSLOT])

        for k in range(NCHUNK):
            if k >= NSLOT:
                wh[k - NSLOT].wait()
            gh[k] = pltpu.async_copy(tab.at[idx_v.at[k]], buf.at[k % NSLOT],
                                     sem_g[k % NSLOT])
            if k >= 1:
                gh[k - 1].wait()
                wh[k - 1] = start_wb(k - 1)
        gh[NCHUNK - 1].wait()
        wh[NCHUNK - 1] = start_wb(NCHUNK - 1)
        for k in range(NCHUNK - NSLOT, NCHUNK):
            wh[k].wait()
    return body


def _make_gather(rows, width):
    mesh = plsc.VectorSubcoreMesh(core_axis_name="c", subcore_axis_name="s")
    return pl.kernel(
        _make_gather_body(width),
        out_type=jax.ShapeDtypeStruct((BATCH, width), jnp.float32),
        mesh=mesh,
        scratch_types=[
            pltpu.VMEM((NCHUNK, CHUNK), jnp.int32),
            pltpu.VMEM((NSLOT, CHUNK, width), jnp.float32),
        ] + [pltpu.SemaphoreType.DMA] * (2 * NSLOT),
        compiler_params=pltpu.CompilerParams(use_tc_tiling_on_sc=False),
    )


@jax.jit
def _sc_gather_u(user_emb, iu):
    return _make_gather(100000, EMB)(user_emb, iu)


@jax.jit
def _sc_gather_c(item_content, ii):
    return _make_gather(100000, CDIM)(item_content, ii)


def _tc_body(u_ref, c_ref, w_ref, b_ref, o_ref):
    meta = lax.dot_general(c_ref[...], w_ref[...],
                           (((1,), (0,)), ((), ())),
                           preferred_element_type=jnp.float32)
    meta = meta + b_ref[...]
    o_ref[...] = jnp.sum(u_ref[...] * meta, axis=1)[None, None, :]


@jax.jit
def _tc_compute(u_g, c_g, Wt5, b5):
    blk = 2048
    grid = BATCH // blk
    out = pl.pallas_call(
        _tc_body,
        grid=(grid,),
        in_specs=[
            pl.BlockSpec((blk, EMB), lambda i: (i, 0)),
            pl.BlockSpec((blk, CDIM), lambda i: (i, 0)),
            pl.BlockSpec((CDIM, EMB), lambda i: (0, 0)),
            pl.BlockSpec((1, EMB), lambda i: (0, 0)),
        ],
        out_specs=pl.BlockSpec((1, 1, blk), lambda i: (i, 0, 0)),
        out_shape=jax.ShapeDtypeStruct((grid, 1, blk), jnp.float32),
    )(u_g, c_g, Wt5, b5)
    return out.reshape(BATCH)


def kernel(batch_u, batch_i, user_emb, item_emb, item_content, W, b):
    iu = batch_u.astype(jnp.int32).reshape(BATCH // CHUNK, CHUNK)
    ii = batch_i.astype(jnp.int32).reshape(BATCH // CHUNK, CHUNK)
    c_g = _sc_gather_c(item_content, ii)
    iu, c_g = lax.optimization_barrier((iu, c_g))
    u_g = _sc_gather_u(user_emb, iu)
    Wt5 = W.T / 5.0
    b5 = (b / 5.0).reshape(1, EMB)
    return _tc_compute(u_g, c_g, Wt5, b5)


# speedup vs baseline: 1.0421x; 1.0142x over previous
import jax
import jax.numpy as jnp
from jax import lax
from jax.experimental import pallas as pl
from jax.experimental.pallas import tpu as pltpu
from jax.experimental.pallas import tpu_sc as plsc

BATCH = 16384
EMB = 64
CDIM = 128
NC = 2
NS = 16
NW = NC * NS
BPW = BATCH // NW
CHUNK = 128
NCHUNK = BPW // CHUNK
NSLOT = 2


def _make_gather_body(width):
    def body(tab, idx_hbm, out, idx_v, buf, sg0, sg1, sw0, sw1):
        sem_g = [sg0, sg1]
        sem_w = [sw0, sw1]
        wid = lax.axis_index("s") * NC + lax.axis_index("c")
        base = wid * BPW
        pltpu.sync_copy(idx_hbm.at[pl.ds(wid * NCHUNK, NCHUNK)], idx_v)
        gh = [None] * NCHUNK
        wh = [None] * NCHUNK

        def start_wb(k):
            return pltpu.async_copy(
                buf.at[k % NSLOT], out.at[pl.ds(base + k * CHUNK, CHUNK)],
                sem_w[k % NSLOT])

        for k in range(NCHUNK):
            if k >= NSLOT:
                wh[k - NSLOT].wait()
            gh[k] = pltpu.async_copy(tab.at[idx_v.at[k]], buf.at[k % NSLOT],
                                     sem_g[k % NSLOT])
            if k >= 1:
                gh[k - 1].wait()
                wh[k - 1] = start_wb(k - 1)
        gh[NCHUNK - 1].wait()
        wh[NCHUNK - 1] = start_wb(NCHUNK - 1)
        for k in range(NCHUNK - NSLOT, NCHUNK):
            wh[k].wait()
    return body


def _make_gather(rows, width):
    mesh = plsc.VectorSubcoreMesh(core_axis_name="c", subcore_axis_name="s")
    return pl.kernel(
        _make_gather_body(width),
        out_type=jax.ShapeDtypeStruct((BATCH, width), jnp.float32),
        mesh=mesh,
        scratch_types=[
            pltpu.VMEM((NCHUNK, CHUNK), jnp.int32),
            pltpu.VMEM((NSLOT, CHUNK, width), jnp.float32),
        ] + [pltpu.SemaphoreType.DMA] * (2 * NSLOT),
        compiler_params=pltpu.CompilerParams(use_tc_tiling_on_sc=False),
    )


@jax.jit
def _sc_gather_u(user_emb, iu):
    return _make_gather(100000, EMB)(user_emb, iu)


@jax.jit
def _sc_gather_c(item_content, ii):
    return _make_gather(100000, CDIM)(item_content, ii)


def _tc_body(u_ref, c_ref, w_ref, b_ref, o_ref):
    meta = lax.dot_general(c_ref[...], w_ref[...],
                           (((1,), (0,)), ((), ())),
                           preferred_element_type=jnp.float32)
    meta = meta + b_ref[...]
    o_ref[...] = jnp.sum(u_ref[...] * meta, axis=1)[None, None, :]


@jax.jit
def _tc_compute(u_g, c_g, Wt5, b5):
    blk = 4096
    grid = BATCH // blk
    out = pl.pallas_call(
        _tc_body,
        grid=(grid,),
        in_specs=[
            pl.BlockSpec((blk, EMB), lambda i: (i, 0)),
            pl.BlockSpec((blk, CDIM), lambda i: (i, 0)),
            pl.BlockSpec((CDIM, EMB), lambda i: (0, 0)),
            pl.BlockSpec((1, EMB), lambda i: (0, 0)),
        ],
        out_specs=pl.BlockSpec((1, 1, blk), lambda i: (i, 0, 0)),
        out_shape=jax.ShapeDtypeStruct((grid, 1, blk), jnp.float32),
    )(u_g, c_g, Wt5, b5)
    return out.reshape(BATCH)


def kernel(batch_u, batch_i, user_emb, item_emb, item_content, W, b):
    iu = batch_u.astype(jnp.int32).reshape(BATCH // CHUNK, CHUNK)
    ii = batch_i.astype(jnp.int32).reshape(BATCH // CHUNK, CHUNK)
    c_g = _sc_gather_c(item_content, ii)
    iu, c_g = lax.optimization_barrier((iu, c_g))
    u_g = _sc_gather_u(user_emb, iu)
    Wt5 = W.T / 5.0
    b5 = (b / 5.0).reshape(1, EMB)
    return _tc_compute(u_g, c_g, Wt5, b5)
